# baseline (speedup 1.0000x reference)
import functools

import jax
import jax.numpy as jnp
from jax import lax
from jax.experimental import pallas as pl
from jax.experimental.pallas import tpu as pltpu, tpu_sc as plsc

N = 10000
E = 320000
D = 128

NC = 2
NS = 16
NW = NC * NS

NPAD = 10240
CHUNK = 64
NBUF = 4
CHUNKS_PER_TILE = 160
EPAD = NW * CHUNKS_PER_TILE * CHUNK
ROWS_PER_TILE = NPAD // NS


def _sc_segment_sum(features, edges3, zrows, zflat, ones_row):
    mesh = plsc.VectorSubcoreMesh(
        core_axis_name="c", subcore_axis_name="s",
        num_cores=NC, num_subcores=NS)

    @functools.partial(
        pl.kernel,
        out_type=(
            jax.ShapeDtypeStruct((NC, NPAD, D), jnp.float32),
            jax.ShapeDtypeStruct((NC, NPAD), jnp.float32),
        ),
        mesh=mesh,
        scratch_types=[
            [pltpu.VMEM((CHUNK,), jnp.int32)] * NBUF,
            [pltpu.VMEM((CHUNK,), jnp.int32)] * NBUF,
            [pltpu.VMEM((CHUNK, D), jnp.float32)] * NBUF,
            pltpu.VMEM((CHUNK,), jnp.float32),
            pltpu.VMEM_SHARED((NPAD, D), jnp.float32),
            pltpu.VMEM_SHARED((NPAD,), jnp.float32),
            [pltpu.SemaphoreType.DMA] * NBUF,
        ],
    )
    def sc_kernel(feat_hbm, e3_hbm, zrows_hbm, zflat_hbm, ones_hbm,
                  h1p_hbm, degp_hbm,
                  src_r, dst_r, rows_r, ones_v, h1_acc, deg_acc, sem_r):
        c = lax.axis_index("c")
        s = lax.axis_index("s")
        tid = c * NS + s
        row0 = s * ROWS_PER_TILE
        chunk0 = tid * CHUNKS_PER_TILE
        last = chunk0 + CHUNKS_PER_TILE - 1

        def fetch(row, b):
            pltpu.sync_copy(e3_hbm.at[0, row], src_r[b])
            pltpu.sync_copy(e3_hbm.at[1, row], dst_r[b])
            pltpu.async_copy(feat_hbm.at[src_r[b]], rows_r[b], sem_r[b])

        def scat(b):
            pltpu.sync_copy(rows_r[b], h1_acc.at[dst_r[b]], add=True)
            pltpu.sync_copy(ones_v, deg_acc.at[dst_r[b]], add=True)

        for b in range(NBUF - 1):
            fetch(chunk0 + b, b)

        pltpu.sync_copy(zrows_hbm.at[pl.ds(row0, ROWS_PER_TILE)],
                        h1_acc.at[pl.ds(row0, ROWS_PER_TILE)])
        pltpu.sync_copy(zflat_hbm.at[pl.ds(row0, ROWS_PER_TILE)],
                        deg_acc.at[pl.ds(row0, ROWS_PER_TILE)])
        pltpu.sync_copy(ones_hbm, ones_v)
        plsc.subcore_barrier()

        def step(i, _):
            cbase = chunk0 + NBUF * i
            for b in range(NBUF):
                pltpu.make_async_copy(
                    feat_hbm.at[src_r[b]], rows_r[b], sem_r[b]).wait()
                nxt = jnp.minimum(cbase + b + NBUF - 1, last)
                fetch(nxt, (b + NBUF - 1) % NBUF)
                scat(b)
            return _

        lax.fori_loop(0, CHUNKS_PER_TILE // NBUF, step, 0)
        for b in range(NBUF - 1):
            pltpu.make_async_copy(
                feat_hbm.at[src_r[b]], rows_r[b], sem_r[b]).wait()
        plsc.subcore_barrier()

        pltpu.sync_copy(h1_acc.at[pl.ds(row0, ROWS_PER_TILE)],
                        h1p_hbm.at[c, pl.ds(row0, ROWS_PER_TILE)])
        pltpu.sync_copy(deg_acc.at[pl.ds(row0, ROWS_PER_TILE)],
                        degp_hbm.at[c, pl.ds(row0, ROWS_PER_TILE)])

    return sc_kernel(features, edges3, zrows, zflat, ones_row)


BN = 1024


def _tc_h0(i_ref, w0t_ref, b0_ref, out_ref):
    x = i_ref[...]
    h0 = jnp.dot(x, w0t_ref[...], preferred_element_type=jnp.float32)
    out_ref[...] = h0 + b0_ref[...]


def _tc_h1(h1p_ref, degp_ref, w1t_ref, b1_ref, out_ref):
    hp = h1p_ref[0, :, :] + h1p_ref[1, :, :]
    dg = degp_ref[0, :] + degp_ref[1, :]
    din = 1.0 / jnp.maximum(dg, 1.0)
    h1 = hp * din[:, None]
    h1o = jnp.dot(h1, w1t_ref[...], preferred_element_type=jnp.float32)
    out_ref[...] = h1o + b1_ref[...]


def kernel(features, edge_index, W0, b0, W1, b1):
    pad = EPAD - E
    epad = jnp.concatenate(
        [jnp.zeros((1, pad), jnp.int32),
         jnp.full((1, pad), NPAD - 1, jnp.int32)], axis=0)
    edges3 = jnp.concatenate([edge_index, epad], axis=1).reshape(2, EPAD // CHUNK, CHUNK)
    zrows = jnp.zeros((NPAD, D), jnp.float32)
    zflat = jnp.zeros((NPAD,), jnp.float32)
    ones_row = jnp.ones((CHUNK,), jnp.float32)

    grid = (NPAD // BN,)
    h0 = pl.pallas_call(
        _tc_h0,
        grid=grid,
        in_specs=[
            pl.BlockSpec((BN, D), lambda i: (i, 0)),
            pl.BlockSpec((D, D), lambda i: (0, 0)),
            pl.BlockSpec((1, D), lambda i: (0, 0)),
        ],
        out_specs=pl.BlockSpec((BN, D), lambda i: (i, 0)),
        out_shape=jax.ShapeDtypeStruct((N, D), jnp.float32),
    )(features, W0.T, b0.reshape(1, D))

    h1p, degp = _sc_segment_sum(features, edges3, zrows, zflat, ones_row)

    h1o = pl.pallas_call(
        _tc_h1,
        grid=grid,
        in_specs=[
            pl.BlockSpec((NC, BN, D), lambda i: (0, i, 0)),
            pl.BlockSpec((NC, BN), lambda i: (0, i)),
            pl.BlockSpec((D, D), lambda i: (0, 0)),
            pl.BlockSpec((1, D), lambda i: (0, 0)),
        ],
        out_specs=pl.BlockSpec((BN, D), lambda i: (i, 0)),
        out_shape=jax.ShapeDtypeStruct((N, D), jnp.float32),
    )(h1p, degp, W1.T, b1.reshape(1, D))
    return jnp.concatenate([h0, h1o], axis=1)


# speedup vs baseline: 1.0108x; 1.0108x over previous
import functools

import jax
import jax.numpy as jnp
from jax import lax
from jax.experimental import pallas as pl
from jax.experimental.pallas import tpu as pltpu, tpu_sc as plsc

N = 10000
E = 320000
D = 128

NC = 2
NS = 16
NW = NC * NS

NPAD = 10240
CHUNK = 64
NBUF = 4
CHUNKS_PER_TILE = 160
EPAD = NW * CHUNKS_PER_TILE * CHUNK
ROWS_PER_TILE = NPAD // NS


def _sc_segment_sum(features, edges3, zrows, zflat, ones_row):
    mesh = plsc.VectorSubcoreMesh(
        core_axis_name="c", subcore_axis_name="s",
        num_cores=NC, num_subcores=NS)

    @functools.partial(
        pl.kernel,
        out_type=(
            jax.ShapeDtypeStruct((NC, NPAD, D), jnp.float32),
            jax.ShapeDtypeStruct((NC, NPAD), jnp.float32),
        ),
        mesh=mesh,
        scratch_types=[
            [pltpu.VMEM((CHUNK,), jnp.int32)] * NBUF,
            [pltpu.VMEM((CHUNK,), jnp.int32)] * NBUF,
            [pltpu.VMEM((CHUNK, D), jnp.float32)] * NBUF,
            pltpu.VMEM((CHUNK,), jnp.float32),
            pltpu.VMEM_SHARED((NPAD, D), jnp.float32),
            pltpu.VMEM_SHARED((NPAD,), jnp.float32),
            [pltpu.SemaphoreType.DMA] * NBUF,
        ],
    )
    def sc_kernel(feat_hbm, e3_hbm, zrows_hbm, zflat_hbm, ones_hbm,
                  h1p_hbm, degp_hbm,
                  src_r, dst_r, rows_r, ones_v, h1_acc, deg_acc, sem_r):
        c = lax.axis_index("c")
        s = lax.axis_index("s")
        tid = c * NS + s
        row0 = s * ROWS_PER_TILE
        chunk0 = tid * CHUNKS_PER_TILE
        last = chunk0 + CHUNKS_PER_TILE - 1

        def fetch(row, b):
            pltpu.sync_copy(e3_hbm.at[0, row], src_r[b])
            pltpu.sync_copy(e3_hbm.at[1, row], dst_r[b])
            pltpu.async_copy(feat_hbm.at[src_r[b]], rows_r[b], sem_r[b])

        def scat(b):
            pltpu.sync_copy(rows_r[b], h1_acc.at[dst_r[b]], add=True)
            pltpu.sync_copy(ones_v, deg_acc.at[dst_r[b]], add=True)

        for b in range(NBUF - 1):
            fetch(chunk0 + b, b)

        pltpu.sync_copy(zrows_hbm.at[pl.ds(row0, ROWS_PER_TILE)],
                        h1_acc.at[pl.ds(row0, ROWS_PER_TILE)])
        pltpu.sync_copy(zflat_hbm.at[pl.ds(row0, ROWS_PER_TILE)],
                        deg_acc.at[pl.ds(row0, ROWS_PER_TILE)])
        pltpu.sync_copy(ones_hbm, ones_v)
        plsc.subcore_barrier()

        def step(i, _):
            cbase = chunk0 + NBUF * i
            for b in range(NBUF):
                pltpu.make_async_copy(
                    feat_hbm.at[src_r[b]], rows_r[b], sem_r[b]).wait()
                nxt = jnp.minimum(cbase + b + NBUF - 1, last)
                fetch(nxt, (b + NBUF - 1) % NBUF)
                scat(b)
            return _

        lax.fori_loop(0, CHUNKS_PER_TILE // NBUF, step, 0)
        for b in range(NBUF - 1):
            pltpu.make_async_copy(
                feat_hbm.at[src_r[b]], rows_r[b], sem_r[b]).wait()
        plsc.subcore_barrier()

        pltpu.sync_copy(h1_acc.at[pl.ds(row0, ROWS_PER_TILE)],
                        h1p_hbm.at[c, pl.ds(row0, ROWS_PER_TILE)])
        pltpu.sync_copy(deg_acc.at[pl.ds(row0, ROWS_PER_TILE)],
                        degp_hbm.at[c, pl.ds(row0, ROWS_PER_TILE)])

    return sc_kernel(features, edges3, zrows, zflat, ones_row)


BN = 1024


def _tc_dense(i_ref, h1p_ref, degp_ref, w0t_ref, b0_ref, w1t_ref, b1_ref,
              out_ref):
    x = i_ref[...]
    h0 = jnp.dot(x, w0t_ref[...], preferred_element_type=jnp.float32)
    h0 = h0 + b0_ref[...]
    hp = h1p_ref[0, :, :] + h1p_ref[1, :, :]
    dg = degp_ref[0, :] + degp_ref[1, :]
    din = 1.0 / jnp.maximum(dg, 1.0)
    h1 = hp * din[:, None]
    h1o = jnp.dot(h1, w1t_ref[...], preferred_element_type=jnp.float32)
    h1o = h1o + b1_ref[...]
    out_ref[...] = jnp.concatenate([h0, h1o], axis=1)


def kernel(features, edge_index, W0, b0, W1, b1):
    pad = EPAD - E
    epad = jnp.concatenate(
        [jnp.zeros((1, pad), jnp.int32),
         jnp.full((1, pad), NPAD - 1, jnp.int32)], axis=0)
    edges3 = jnp.concatenate([edge_index, epad], axis=1).reshape(2, EPAD // CHUNK, CHUNK)
    zrows = jnp.zeros((NPAD, D), jnp.float32)
    zflat = jnp.zeros((NPAD,), jnp.float32)
    ones_row = jnp.ones((CHUNK,), jnp.float32)

    h1p, degp = _sc_segment_sum(features, edges3, zrows, zflat, ones_row)

    grid = (NPAD // BN,)
    out = pl.pallas_call(
        _tc_dense,
        grid=grid,
        in_specs=[
            pl.BlockSpec((BN, D), lambda i: (i, 0)),
            pl.BlockSpec((NC, BN, D), lambda i: (0, i, 0)),
            pl.BlockSpec((NC, BN), lambda i: (0, i)),
            pl.BlockSpec((D, D), lambda i: (0, 0)),
            pl.BlockSpec((1, D), lambda i: (0, 0)),
            pl.BlockSpec((D, D), lambda i: (0, 0)),
            pl.BlockSpec((1, D), lambda i: (0, 0)),
        ],
        out_specs=pl.BlockSpec((BN, 2 * D), lambda i: (i, 0)),
        out_shape=jax.ShapeDtypeStruct((N, 2 * D), jnp.float32),
    )(features, h1p, degp, W0.T, b0.reshape(1, D), W1.T, b1.reshape(1, D))
    return out
